# baseline (speedup 1.0000x reference)
import functools

import jax
import jax.numpy as jnp
from jax import lax
from jax.experimental import pallas as pl
from jax.experimental.pallas import tpu as pltpu
from jax.experimental.pallas import tpu_sc as plsc

NUM_STEPS = 1000
EMB_DIM = 128
BATCH = 16384

_NC = 2
_NS = 16
_NW = _NC * _NS
_BPW = BATCH // _NW
_CHUNK = 64
_NCH = _BPW // _CHUNK
_HBM_CH = 2


_TBL = 1024


def _mlp_body(emb_ref, w1_ref, b1_ref, w2_ref, b2_ref, o_ref):
    x = jnp.concatenate(
        [emb_ref[...], jnp.zeros((_TBL - NUM_STEPS, EMB_DIM), jnp.float32)],
        axis=0)
    h = lax.dot_general(x, w1_ref[...], (((1,), (1,)), ((), ())),
                        preferred_element_type=jnp.float32) + b1_ref[...]
    h = h * jax.nn.sigmoid(h)
    g = lax.dot_general(h, w2_ref[...], (((1,), (1,)), ((), ())),
                        preferred_element_type=jnp.float32) + b2_ref[...]
    o_ref[...] = g * jax.nn.sigmoid(g)


def _mlp_table(embedding, W1, b1, W2, b2):
    return pl.pallas_call(
        _mlp_body,
        out_shape=jax.ShapeDtypeStruct((_TBL, EMB_DIM), jnp.float32),
    )(embedding, W1, b1.reshape(1, EMB_DIM), W2, b2.reshape(1, EMB_DIM))


@functools.cache
def _make_sc_gather():
    mesh = plsc.VectorSubcoreMesh(core_axis_name="c", subcore_axis_name="s")

    @functools.partial(
        pl.kernel,
        out_type=jax.ShapeDtypeStruct((BATCH, EMB_DIM), jnp.float32),
        mesh=mesh,
        scratch_types=[
            pltpu.VMEM((_NCH, _CHUNK), jnp.int32),
            pltpu.VMEM((_BPW, EMB_DIM), jnp.float32),
            pltpu.MemorySpace.VMEM_SHARED((_TBL, EMB_DIM), jnp.float32),
            pltpu.SemaphoreType.DMA,
            pltpu.SemaphoreType.DMA,
            pltpu.SemaphoreType.DMA,
            pltpu.SemaphoreType.DMA,
        ],
    )
    def _sc_gather(table_hbm, idx_hbm, out_hbm, idx_v, rows_v,
                   shared, sem_i, sem_t, sem_g, sem_w):
        wid = lax.axis_index("s") * _NC + lax.axis_index("c")
        sid = lax.axis_index("s")
        rows_per_tile = _TBL // _NS
        idx_cp = pltpu.async_copy(
            idx_hbm.at[pl.ds(wid * _NCH, _NCH)], idx_v, sem_i)
        stage_cp = pltpu.async_copy(
            table_hbm.at[pl.ds(sid * rows_per_tile, rows_per_tile)],
            shared.at[pl.ds(sid * rows_per_tile, rows_per_tile)],
            sem_t)
        idx_cp.wait()
        head_gathers = [
            pltpu.async_copy(
                table_hbm.at[idx_v.at[j]],
                rows_v.at[pl.ds(j * _CHUNK, _CHUNK)],
                sem_i,
            )
            for j in range(_HBM_CH)
        ]
        stage_cp.wait()
        plsc.subcore_barrier()
        tail_gathers = [
            pltpu.async_copy(
                shared.at[idx_v.at[j]],
                rows_v.at[pl.ds(j * _CHUNK, _CHUNK)],
                sem_g,
            )
            for j in range(_HBM_CH, _NCH)
        ]
        writes = []
        for j in range(_NCH):
            (head_gathers[j] if j < _HBM_CH else tail_gathers[j - _HBM_CH]).wait()
            writes.append(
                pltpu.async_copy(
                    rows_v.at[pl.ds(j * _CHUNK, _CHUNK)],
                    out_hbm.at[pl.ds(wid * _BPW + j * _CHUNK, _CHUNK)],
                    sem_w,
                )
            )
        for w in writes:
            w.wait()

    return _sc_gather


def kernel(diffusion_step, embedding, W1, b1, W2, b2):
    table = _mlp_table(embedding, W1, b1, W2, b2)
    idx = diffusion_step.astype(jnp.int32).reshape(BATCH // _CHUNK, _CHUNK)
    return _make_sc_gather()(table, idx)


# speedup vs baseline: 1.0832x; 1.0832x over previous
import functools

import jax
import jax.numpy as jnp
from jax import lax
from jax.experimental import pallas as pl
from jax.experimental.pallas import tpu as pltpu
from jax.experimental.pallas import tpu_sc as plsc

NUM_STEPS = 1000
EMB_DIM = 128
BATCH = 16384

_NC = 2
_NS = 16
_NW = _NC * _NS
_BPW = BATCH // _NW
_CHUNK = 128
_NCH = _BPW // _CHUNK


_TBL = 1024


def _mlp_body(emb_ref, w1_ref, b1_ref, w2_ref, b2_ref, o_ref):
    x = jnp.concatenate(
        [emb_ref[...], jnp.zeros((_TBL - NUM_STEPS, EMB_DIM), jnp.float32)],
        axis=0)
    h = lax.dot_general(x, w1_ref[...], (((1,), (1,)), ((), ())),
                        preferred_element_type=jnp.float32) + b1_ref[...]
    h = h * jax.nn.sigmoid(h)
    g = lax.dot_general(h, w2_ref[...], (((1,), (1,)), ((), ())),
                        preferred_element_type=jnp.float32) + b2_ref[...]
    o_ref[...] = g * jax.nn.sigmoid(g)


def _mlp_table(embedding, W1, b1, W2, b2):
    return pl.pallas_call(
        _mlp_body,
        out_shape=jax.ShapeDtypeStruct((_TBL, EMB_DIM), jnp.float32),
    )(embedding, W1, b1.reshape(1, EMB_DIM), W2, b2.reshape(1, EMB_DIM))


@functools.cache
def _make_sc_gather():
    mesh = plsc.VectorSubcoreMesh(core_axis_name="c", subcore_axis_name="s")

    @functools.partial(
        pl.kernel,
        out_type=jax.ShapeDtypeStruct((BATCH, EMB_DIM), jnp.float32),
        mesh=mesh,
        scratch_types=[
            pltpu.VMEM((_NCH, _CHUNK), jnp.int32),
            pltpu.VMEM((_BPW, EMB_DIM), jnp.float32),
            pltpu.MemorySpace.VMEM_SHARED((_TBL, EMB_DIM), jnp.float32),
            pltpu.SemaphoreType.DMA,
            pltpu.SemaphoreType.DMA,
            pltpu.SemaphoreType.DMA,
            pltpu.SemaphoreType.DMA,
        ],
    )
    def _sc_gather(table_hbm, idx_hbm, out_hbm, idx_v, rows_v,
                   shared, sem_i, sem_t, sem_g, sem_w):
        wid = lax.axis_index("s") * _NC + lax.axis_index("c")
        sid = lax.axis_index("s")
        rows_per_tile = _TBL // _NS
        idx_cp = pltpu.async_copy(
            idx_hbm.at[pl.ds(wid * _NCH, _NCH)], idx_v, sem_i)
        stage_cp = pltpu.async_copy(
            table_hbm.at[pl.ds(sid * rows_per_tile, rows_per_tile)],
            shared.at[pl.ds(sid * rows_per_tile, rows_per_tile)],
            sem_t)
        idx_cp.wait()
        g0 = pltpu.async_copy(
            table_hbm.at[idx_v.at[0]], rows_v.at[pl.ds(0, _CHUNK)], sem_i)
        stage_cp.wait()
        plsc.subcore_barrier()
        gathers = [
            pltpu.async_copy(
                shared.at[idx_v.at[j]],
                rows_v.at[pl.ds(j * _CHUNK, _CHUNK)],
                sem_g,
            )
            for j in range(1, _NCH)
        ]
        writes = []
        g0.wait()
        writes.append(
            pltpu.async_copy(
                rows_v.at[pl.ds(0, _CHUNK)],
                out_hbm.at[pl.ds(wid * _BPW, _CHUNK)],
                sem_w,
            )
        )
        for j in range(1, _NCH):
            gathers[j - 1].wait()
            writes.append(
                pltpu.async_copy(
                    rows_v.at[pl.ds(j * _CHUNK, _CHUNK)],
                    out_hbm.at[pl.ds(wid * _BPW + j * _CHUNK, _CHUNK)],
                    sem_w,
                )
            )
        for w in writes:
            w.wait()

    return _sc_gather


def kernel(diffusion_step, embedding, W1, b1, W2, b2):
    table = _mlp_table(embedding, W1, b1, W2, b2)
    idx = diffusion_step.astype(jnp.int32).reshape(BATCH // _CHUNK, _CHUNK)
    return _make_sc_gather()(table, idx)


# speedup vs baseline: 1.0858x; 1.0024x over previous
import functools

import jax
import jax.numpy as jnp
from jax import lax
from jax.experimental import pallas as pl
from jax.experimental.pallas import tpu as pltpu
from jax.experimental.pallas import tpu_sc as plsc

NUM_STEPS = 1000
EMB_DIM = 128
BATCH = 16384

_NC = 2
_NS = 16
_NW = _NC * _NS
_BPW = BATCH // _NW
_CHUNK = 128
_NCH = _BPW // _CHUNK


_TBL = 1024


def _mlp_body(emb_ref, w1_ref, b1_ref, w2_ref, b2_ref, o_ref):
    x = jnp.concatenate(
        [emb_ref[...], jnp.zeros((_TBL - NUM_STEPS, EMB_DIM), jnp.float32)],
        axis=0)
    h = lax.dot_general(x, w1_ref[...], (((1,), (1,)), ((), ())),
                        preferred_element_type=jnp.float32) + b1_ref[...]
    h = h * jax.nn.sigmoid(h)
    g = lax.dot_general(h, w2_ref[...], (((1,), (1,)), ((), ())),
                        preferred_element_type=jnp.float32) + b2_ref[...]
    o_ref[...] = g * jax.nn.sigmoid(g)


def _mlp_table(embedding, W1, b1, W2, b2):
    return pl.pallas_call(
        _mlp_body,
        out_shape=jax.ShapeDtypeStruct((_TBL, EMB_DIM), jnp.float32),
    )(embedding, W1, b1.reshape(1, EMB_DIM), W2, b2.reshape(1, EMB_DIM))


@functools.cache
def _make_sc_gather():
    mesh = plsc.VectorSubcoreMesh(core_axis_name="c", subcore_axis_name="s")

    @functools.partial(
        pl.kernel,
        out_type=jax.ShapeDtypeStruct((BATCH, EMB_DIM), jnp.float32),
        mesh=mesh,
        scratch_types=[
            pltpu.VMEM((_NCH, _CHUNK), jnp.int32),
            pltpu.VMEM((_BPW, EMB_DIM), jnp.float32),
            pltpu.MemorySpace.VMEM_SHARED((_TBL, EMB_DIM), jnp.float32),
            pltpu.SemaphoreType.DMA,
            pltpu.SemaphoreType.DMA,
            pltpu.SemaphoreType.DMA((_NCH,)),
            pltpu.SemaphoreType.DMA,
        ],
    )
    def _sc_gather(table_hbm, idx_hbm, out_hbm, idx_v, rows_v,
                   shared, sem_i, sem_t, sem_g, sem_w):
        wid = lax.axis_index("s") * _NC + lax.axis_index("c")
        sid = lax.axis_index("s")
        rows_per_tile = _TBL // _NS
        idx_cp = pltpu.async_copy(
            idx_hbm.at[pl.ds(wid * _NCH, _NCH)], idx_v, sem_i)
        stage_cp = pltpu.async_copy(
            table_hbm.at[pl.ds(sid * rows_per_tile, rows_per_tile)],
            shared.at[pl.ds(sid * rows_per_tile, rows_per_tile)],
            sem_t)
        idx_cp.wait()
        g0 = pltpu.async_copy(
            table_hbm.at[idx_v.at[0]], rows_v.at[pl.ds(0, _CHUNK)], sem_i)
        stage_cp.wait()
        plsc.subcore_barrier()
        gathers = [
            pltpu.async_copy(
                shared.at[idx_v.at[j]],
                rows_v.at[pl.ds(j * _CHUNK, _CHUNK)],
                sem_g.at[j],
            )
            for j in range(1, _NCH)
        ]
        writes = []
        g0.wait()
        writes.append(
            pltpu.async_copy(
                rows_v.at[pl.ds(0, _CHUNK)],
                out_hbm.at[pl.ds(wid * _BPW, _CHUNK)],
                sem_w,
            )
        )
        for j in range(1, _NCH):
            gathers[j - 1].wait()
            writes.append(
                pltpu.async_copy(
                    rows_v.at[pl.ds(j * _CHUNK, _CHUNK)],
                    out_hbm.at[pl.ds(wid * _BPW + j * _CHUNK, _CHUNK)],
                    sem_w,
                )
            )
        for w in writes:
            w.wait()

    return _sc_gather


def kernel(diffusion_step, embedding, W1, b1, W2, b2):
    table = _mlp_table(embedding, W1, b1, W2, b2)
    idx = diffusion_step.astype(jnp.int32).reshape(BATCH // _CHUNK, _CHUNK)
    return _make_sc_gather()(table, idx)


# speedup vs baseline: 1.0974x; 1.0107x over previous
import functools

import jax
import jax.numpy as jnp
from jax import lax
from jax.experimental import pallas as pl
from jax.experimental.pallas import tpu as pltpu
from jax.experimental.pallas import tpu_sc as plsc

NUM_STEPS = 1000
EMB_DIM = 128
BATCH = 16384

_NC = 2
_NS = 16
_NW = _NC * _NS
_BPW = BATCH // _NW
_CHUNK = 128
_NCH = _BPW // _CHUNK


_TBL = 1024


def _mlp_body(emb_ref, w1_ref, b1_ref, w2_ref, b2_ref, o_ref):
    x = emb_ref[...]
    h = lax.dot_general(x, w1_ref[...], (((1,), (1,)), ((), ())),
                        preferred_element_type=jnp.float32) + b1_ref[...]
    h = h * jax.nn.sigmoid(h)
    g = lax.dot_general(h, w2_ref[...], (((1,), (1,)), ((), ())),
                        preferred_element_type=jnp.float32) + b2_ref[...]
    o_ref[pl.ds(0, NUM_STEPS), :] = g * jax.nn.sigmoid(g)


def _mlp_table(embedding, W1, b1, W2, b2):
    return pl.pallas_call(
        _mlp_body,
        out_shape=jax.ShapeDtypeStruct((_TBL, EMB_DIM), jnp.float32),
    )(embedding, W1, b1.reshape(1, EMB_DIM), W2, b2.reshape(1, EMB_DIM))


@functools.cache
def _make_sc_gather():
    mesh = plsc.VectorSubcoreMesh(core_axis_name="c", subcore_axis_name="s")

    @functools.partial(
        pl.kernel,
        out_type=jax.ShapeDtypeStruct((BATCH, EMB_DIM), jnp.float32),
        mesh=mesh,
        scratch_types=[
            pltpu.VMEM((_NCH, _CHUNK), jnp.int32),
            pltpu.VMEM((_BPW, EMB_DIM), jnp.float32),
            pltpu.MemorySpace.VMEM_SHARED((_TBL, EMB_DIM), jnp.float32),
            pltpu.SemaphoreType.DMA,
            pltpu.SemaphoreType.DMA,
            pltpu.SemaphoreType.DMA((_NCH,)),
            pltpu.SemaphoreType.DMA,
        ],
    )
    def _sc_gather(table_hbm, idx_hbm, out_hbm, idx_v, rows_v,
                   shared, sem_i, sem_t, sem_g, sem_w):
        wid = lax.axis_index("s") * _NC + lax.axis_index("c")
        sid = lax.axis_index("s")
        rows_per_tile = _TBL // _NS
        idx_cp = pltpu.async_copy(
            idx_hbm.at[pl.ds(wid * _NCH, _NCH)], idx_v, sem_i)
        stage_cp = pltpu.async_copy(
            table_hbm.at[pl.ds(sid * rows_per_tile, rows_per_tile)],
            shared.at[pl.ds(sid * rows_per_tile, rows_per_tile)],
            sem_t)
        idx_cp.wait()
        g0 = pltpu.async_copy(
            table_hbm.at[idx_v.at[0]], rows_v.at[pl.ds(0, _CHUNK)], sem_i)
        stage_cp.wait()
        plsc.subcore_barrier()
        gathers = [
            pltpu.async_copy(
                shared.at[idx_v.at[j]],
                rows_v.at[pl.ds(j * _CHUNK, _CHUNK)],
                sem_g.at[j],
            )
            for j in range(1, _NCH)
        ]
        writes = []
        g0.wait()
        writes.append(
            pltpu.async_copy(
                rows_v.at[pl.ds(0, _CHUNK)],
                out_hbm.at[pl.ds(wid * _BPW, _CHUNK)],
                sem_w,
            )
        )
        for j in range(1, _NCH):
            gathers[j - 1].wait()
            writes.append(
                pltpu.async_copy(
                    rows_v.at[pl.ds(j * _CHUNK, _CHUNK)],
                    out_hbm.at[pl.ds(wid * _BPW + j * _CHUNK, _CHUNK)],
                    sem_w,
                )
            )
        for w in writes:
            w.wait()

    return _sc_gather


def kernel(diffusion_step, embedding, W1, b1, W2, b2):
    table = _mlp_table(embedding, W1, b1, W2, b2)
    idx = diffusion_step.astype(jnp.int32).reshape(BATCH // _CHUNK, _CHUNK)
    return _make_sc_gather()(table, idx)
